# Initial kernel scaffold; baseline (speedup 1.0000x reference)
import jax
import jax.numpy as jnp
from jax.experimental import pallas as pl


def kernel(x, edge_index, Wp, bp, W0, a0, W1, a1, Wh1, bh1, Wh2, bh2):
    raise NotImplementedError("write your pallas kernel here")


# speedup vs baseline: 4.3621x; 4.3621x over previous
import functools

import jax
import jax.numpy as jnp
from jax import lax
from jax.experimental import pallas as pl
from jax.experimental.pallas import tpu as pltpu
from jax.experimental.pallas import tpu_sc as plsc

_NS = 16
_L = 16


def _elu(v):
    return jnp.where(v > 0, v, jnp.exp(v) - 1.0)


def _proj_body(x_ref, wt_ref, b_ref, o_ref):
    p = jnp.dot(x_ref[...], wt_ref[...], preferred_element_type=jnp.float32)
    p = p + b_ref[...]
    hh = p.shape[1] // 2
    o_ref[0] = p[:, :hh]
    o_ref[1] = p[:, hh:]


def _layer_mm_body(pa_ref, pb_ref, w_ref, a2_ref, ht_ref, s2_ref):
    hact = _elu(jnp.concatenate([pa_ref[0], pb_ref[0]], axis=-1))
    ht = jnp.dot(hact, w_ref[...], preferred_element_type=jnp.float32)
    hh = ht.shape[1] // 2
    ht_ref[0] = ht[:, :hh]
    ht_ref[1] = ht[:, hh:]
    s2_ref[...] = jnp.dot(ht, a2_ref[...], preferred_element_type=jnp.float32)


def _head_body(pa_ref, pb_ref, w1t_ref, b1_ref, w2t_ref, b2_ref, o_ref):
    hact = _elu(jnp.concatenate([pa_ref[0], pb_ref[0]], axis=-1))
    z = jnp.dot(hact, w1t_ref[...], preferred_element_type=jnp.float32)
    z = jnp.maximum(z + b1_ref[...], 0.0)
    o = jnp.dot(z, w2t_ref[...], preferred_element_type=jnp.float32)
    o_ref[...] = o + b2_ref[...]


@functools.lru_cache(maxsize=None)
def _make_edge_kernel(n, e, hh):
    ept = e // _NS
    nzr = n // _NS
    chk = 80
    cs = 2000
    nst = ept // cs
    ng = ept // _L
    assert n % _NS == 0 and ept % cs == 0 and cs % chk == 0 and cs % _L == 0

    mesh = plsc.VectorSubcoreMesh(
        core_axis_name="c", subcore_axis_name="s",
        num_cores=2, num_subcores=_NS)

    @functools.partial(
        pl.kernel,
        out_type=jax.ShapeDtypeStruct((2 * n, hh), jnp.float32),
        mesh=mesh,
        compiler_params=pltpu.CompilerParams(
            use_tc_tiling_on_sc=False, needs_layout_passes=False),
        scratch_types=[
            pltpu.VMEM((2 * n,), jnp.float32),
            pltpu.VMEM((ept,), jnp.float32),
            pltpu.VMEM((cs,), jnp.int32),
            pltpu.VMEM((cs,), jnp.int32),
            pltpu.VMEM((chk, hh), jnp.float32),
            pltpu.VMEM((chk,), jnp.int32),
            pltpu.VMEM((chk,), jnp.int32),
            pltpu.VMEM((_L,), jnp.float32),
            pltpu.VMEM((_NS, _L), jnp.float32),
            pltpu.VMEM_SHARED((_NS, _L), jnp.float32),
            pltpu.VMEM_SHARED((_NS, _L), jnp.float32),
            pltpu.VMEM_SHARED((n, hh), jnp.float32),
        ],
    )
    def edge_k(ht_hbm, s2_hbm, eix_hbm, zeros_hbm, out_hbm,
               s2_v, w_v, rbig, cbig, gbuf, ridx, cidx, statb, red_v,
               stage_mx, stage_sm, agg):
        c = lax.axis_index("c")
        s = lax.axis_index("s")
        ebase = s * ept

        pltpu.sync_copy(zeros_hbm.at[pl.ds(s * nzr, nzr)],
                        agg.at[pl.ds(s * nzr, nzr)])

        pltpu.sync_copy(s2_hbm, s2_v)

        o16 = jnp.ones((_L,), jnp.int32)
        noff = c * n

        def stage_a(t, mx):
            pltpu.sync_copy(eix_hbm.at[0, pl.ds(ebase + t * cs, cs)], rbig)
            pltpu.sync_copy(eix_hbm.at[1, pl.ds(ebase + t * cs, cs)], cbig)

            def body_a(g, mx2):
                sl = pl.ds(g * _L, _L)
                sr = plsc.load_gather(s2_v, [rbig[sl] * 2])
                sc = plsc.load_gather(s2_v, [cbig[sl] * 2 + o16])
                al = sr + sc
                al = jnp.where(al > 0, al, al * 0.01)
                w_v[pl.ds(t * cs + g * _L, _L)] = al
                return jnp.maximum(mx2, al)

            return lax.fori_loop(0, cs // _L, body_a, mx)

        mx = lax.fori_loop(0, nst, stage_a,
                           jnp.full((_L,), -3.0e38, jnp.float32))
        statb[...] = mx
        pltpu.sync_copy(statb, stage_mx.at[s])
        plsc.subcore_barrier()
        pltpu.sync_copy(stage_mx, red_v)
        acc = red_v[0]
        for i in range(1, _NS):
            acc = jnp.maximum(acc, red_v[i])
        m = jnp.max(acc)

        def body_b(g, sm):
            sl = pl.ds(g * _L, _L)
            ex = jnp.exp(w_v[sl] - m)
            w_v[sl] = ex
            return sm + ex

        sm = lax.fori_loop(0, ng, body_b, jnp.zeros((_L,), jnp.float32))
        statb[...] = sm
        pltpu.sync_copy(statb, stage_sm.at[s])
        plsc.subcore_barrier()
        pltpu.sync_copy(stage_sm, red_v)
        acc = red_v[0]
        for i in range(1, _NS):
            acc = acc + red_v[i]
        invv = jnp.ones((_L,), jnp.float32) / jnp.broadcast_to(
            jnp.sum(acc), (_L,))

        def body_c(g, carry):
            sl = pl.ds(g * _L, _L)
            w_v[sl] = w_v[sl] * invv
            return carry

        lax.fori_loop(0, ng, body_c, 0)

        def stage_c(t, carry):
            pltpu.sync_copy(eix_hbm.at[0, pl.ds(ebase + t * cs, cs)], rbig)
            pltpu.sync_copy(eix_hbm.at[1, pl.ds(ebase + t * cs, cs)], cbig)

            def chunk(k, carry1):
                off = k * chk
                for q in range(chk // _L):
                    sl = pl.ds(q * _L, _L)
                    ridx[sl] = rbig[pl.ds(off + q * _L, _L)]
                    cidx[sl] = cbig[pl.ds(off + q * _L, _L)] + noff
                pltpu.sync_copy(ht_hbm.at[cidx], gbuf)
                woff = t * cs + off

                def rowgrp(rg, carry2):
                    base = rg * 8
                    for u in range(8):
                        r = base + u
                        wv = plsc.load_gather(
                            w_v, [jnp.full((_L,), woff + r, jnp.int32)])
                        for j in range(hh // _L):
                            sl = pl.ds(j * _L, _L)
                            gbuf[r, sl] = gbuf[r, sl] * wv
                    return carry2

                lax.fori_loop(0, chk // 8, rowgrp, 0)
                pltpu.sync_copy(gbuf, agg.at[ridx], add=True)
                return carry1

            lax.fori_loop(0, cs // chk, chunk, 0)
            return carry

        lax.fori_loop(0, nst, stage_c, 0)

        plsc.subcore_barrier()
        pltpu.sync_copy(agg.at[pl.ds(s * nzr, nzr)],
                        out_hbm.at[pl.ds(c * n + s * nzr, nzr)])

    return edge_k


def kernel(x, edge_index, Wp, bp, W0, a0, W1, a1, Wh1, bh1, Wh2, bh2):
    n, d = x.shape
    e = edge_index.shape[1]
    h = Wp.shape[0]
    hh = h // 2
    co = Wh2.shape[0]
    bn = 1000 if n % 1000 == 0 else n
    grid = (n // bn,)
    f32 = jnp.float32

    p0 = pl.pallas_call(
        _proj_body,
        grid=grid,
        in_specs=[
            pl.BlockSpec((bn, d), lambda i: (i, 0)),
            pl.BlockSpec((d, h), lambda i: (0, 0)),
            pl.BlockSpec((1, h), lambda i: (0, 0)),
        ],
        out_specs=pl.BlockSpec((2, bn, hh), lambda i: (0, i, 0)),
        out_shape=jax.ShapeDtypeStruct((2, n, hh), f32),
    )(x, Wp.T, bp[None, :])

    edge_k = _make_edge_kernel(n, e, hh)
    zeros_h = jnp.zeros((n, hh), f32)

    def run_layer(p, W, a):
        a2 = jnp.stack([a[:h], a[h:]], axis=1)
        ht2, s2 = pl.pallas_call(
            _layer_mm_body,
            grid=grid,
            in_specs=[
                pl.BlockSpec((1, bn, hh), lambda i: (0, i, 0)),
                pl.BlockSpec((1, bn, hh), lambda i: (1, i, 0)),
                pl.BlockSpec((h, h), lambda i: (0, 0)),
                pl.BlockSpec((h, 2), lambda i: (0, 0)),
            ],
            out_specs=[
                pl.BlockSpec((2, bn, hh), lambda i: (0, i, 0)),
                pl.BlockSpec((bn, 2), lambda i: (i, 0)),
            ],
            out_shape=[
                jax.ShapeDtypeStruct((2, n, hh), f32),
                jax.ShapeDtypeStruct((n, 2), f32),
            ],
        )(p, p, W, a2)
        agg_flat = edge_k(ht2.reshape(2 * n, hh), s2.reshape(2 * n),
                          edge_index, zeros_h)
        return agg_flat.reshape(2, n, hh)

    p1 = run_layer(p0, W0, a0)
    p2 = run_layer(p1, W1, a1)

    out = pl.pallas_call(
        _head_body,
        grid=grid,
        in_specs=[
            pl.BlockSpec((1, bn, hh), lambda i: (0, i, 0)),
            pl.BlockSpec((1, bn, hh), lambda i: (1, i, 0)),
            pl.BlockSpec((h, hh), lambda i: (0, 0)),
            pl.BlockSpec((1, hh), lambda i: (0, 0)),
            pl.BlockSpec((hh, co), lambda i: (0, 0)),
            pl.BlockSpec((1, co), lambda i: (0, 0)),
        ],
        out_specs=pl.BlockSpec((bn, co), lambda i: (i, 0)),
        out_shape=jax.ShapeDtypeStruct((n, co), f32),
    )(p2, p2, Wh1.T, bh1[None, :], Wh2.T, bh2[None, :])
    return out


# speedup vs baseline: 4.4312x; 1.0158x over previous
import functools

import jax
import jax.numpy as jnp
from jax import lax
from jax.experimental import pallas as pl
from jax.experimental.pallas import tpu as pltpu
from jax.experimental.pallas import tpu_sc as plsc

_NS = 16
_L = 16


def _elu(v):
    return jnp.where(v > 0, v, jnp.exp(v) - 1.0)


def _fused0_body(x_ref, wpt_ref, bp_ref, w_ref, a2_ref, ht_ref, s2_ref):
    p = jnp.dot(x_ref[...], wpt_ref[...], preferred_element_type=jnp.float32)
    hact = _elu(p + bp_ref[...])
    ht = jnp.dot(hact, w_ref[...], preferred_element_type=jnp.float32)
    hh = ht.shape[1] // 2
    ht_ref[0] = ht[:, :hh]
    ht_ref[1] = ht[:, hh:]
    s2_ref[...] = jnp.dot(ht, a2_ref[...], preferred_element_type=jnp.float32)


def _layer_mm_body(pa_ref, pb_ref, w_ref, a2_ref, ht_ref, s2_ref):
    hact = _elu(jnp.concatenate([pa_ref[0], pb_ref[0]], axis=-1))
    ht = jnp.dot(hact, w_ref[...], preferred_element_type=jnp.float32)
    hh = ht.shape[1] // 2
    ht_ref[0] = ht[:, :hh]
    ht_ref[1] = ht[:, hh:]
    s2_ref[...] = jnp.dot(ht, a2_ref[...], preferred_element_type=jnp.float32)


def _head_body(pa_ref, pb_ref, w1t_ref, b1_ref, w2t_ref, b2_ref, o_ref):
    hact = _elu(jnp.concatenate([pa_ref[0], pb_ref[0]], axis=-1))
    z = jnp.dot(hact, w1t_ref[...], preferred_element_type=jnp.float32)
    z = jnp.maximum(z + b1_ref[...], 0.0)
    o = jnp.dot(z, w2t_ref[...], preferred_element_type=jnp.float32)
    o_ref[...] = o + b2_ref[...]


@functools.lru_cache(maxsize=None)
def _make_edge_kernel(n, e, hh):
    ept = e // _NS
    nzr = n // _NS
    chk = 80
    cs = 2000
    nst = ept // cs
    ng = ept // _L
    assert n % _NS == 0 and ept % cs == 0 and cs % chk == 0 and cs % _L == 0

    mesh = plsc.VectorSubcoreMesh(
        core_axis_name="c", subcore_axis_name="s",
        num_cores=2, num_subcores=_NS)

    @functools.partial(
        pl.kernel,
        out_type=jax.ShapeDtypeStruct((2 * n, hh), jnp.float32),
        mesh=mesh,
        compiler_params=pltpu.CompilerParams(
            use_tc_tiling_on_sc=False, needs_layout_passes=False),
        scratch_types=[
            pltpu.VMEM((2 * n,), jnp.float32),
            pltpu.VMEM((ept,), jnp.float32),
            pltpu.VMEM((cs,), jnp.int32),
            pltpu.VMEM((cs,), jnp.int32),
            pltpu.VMEM((chk, hh), jnp.float32),
            pltpu.VMEM((chk,), jnp.int32),
            pltpu.VMEM((chk,), jnp.int32),
            pltpu.VMEM((_L,), jnp.float32),
            pltpu.VMEM((_NS, _L), jnp.float32),
            pltpu.VMEM_SHARED((_NS, _L), jnp.float32),
            pltpu.VMEM_SHARED((_NS, _L), jnp.float32),
            pltpu.VMEM_SHARED((n, hh), jnp.float32),
        ],
    )
    def edge_k(ht_hbm, s2_hbm, eix_hbm, zeros_hbm, out_hbm,
               s2_v, w_v, rbig, cbig, gbuf, ridx, cidx, statb, red_v,
               stage_mx, stage_sm, agg):
        c = lax.axis_index("c")
        s = lax.axis_index("s")
        ebase = s * ept

        pltpu.sync_copy(zeros_hbm.at[pl.ds(s * nzr, nzr)],
                        agg.at[pl.ds(s * nzr, nzr)])

        pltpu.sync_copy(s2_hbm, s2_v)

        o16 = jnp.ones((_L,), jnp.int32)
        noff = c * n

        def stage_a(t, mx):
            pltpu.sync_copy(eix_hbm.at[0, pl.ds(ebase + t * cs, cs)], rbig)
            pltpu.sync_copy(eix_hbm.at[1, pl.ds(ebase + t * cs, cs)], cbig)

            def body_a(g, mx2):
                sl = pl.ds(g * _L, _L)
                sr = plsc.load_gather(s2_v, [rbig[sl] * 2])
                sc = plsc.load_gather(s2_v, [cbig[sl] * 2 + o16])
                al = sr + sc
                al = jnp.where(al > 0, al, al * 0.01)
                w_v[pl.ds(t * cs + g * _L, _L)] = al
                return jnp.maximum(mx2, al)

            return lax.fori_loop(0, cs // _L, body_a, mx)

        mx = lax.fori_loop(0, nst, stage_a,
                           jnp.full((_L,), -3.0e38, jnp.float32))
        statb[...] = mx
        pltpu.sync_copy(statb, stage_mx.at[s])
        plsc.subcore_barrier()
        pltpu.sync_copy(stage_mx, red_v)
        acc = red_v[0]
        for i in range(1, _NS):
            acc = jnp.maximum(acc, red_v[i])
        m = jnp.max(acc)

        def body_b(g, sm):
            sl = pl.ds(g * _L, _L)
            ex = jnp.exp(w_v[sl] - m)
            w_v[sl] = ex
            return sm + ex

        sm = lax.fori_loop(0, ng, body_b, jnp.zeros((_L,), jnp.float32))
        statb[...] = sm
        pltpu.sync_copy(statb, stage_sm.at[s])
        plsc.subcore_barrier()
        pltpu.sync_copy(stage_sm, red_v)
        acc = red_v[0]
        for i in range(1, _NS):
            acc = acc + red_v[i]
        invv = jnp.ones((_L,), jnp.float32) / jnp.broadcast_to(
            jnp.sum(acc), (_L,))

        def body_c(g, carry):
            sl = pl.ds(g * _L, _L)
            w_v[sl] = w_v[sl] * invv
            return carry

        lax.fori_loop(0, ng, body_c, 0)

        def stage_c(t, carry):
            pltpu.sync_copy(eix_hbm.at[0, pl.ds(ebase + t * cs, cs)], rbig)
            pltpu.sync_copy(eix_hbm.at[1, pl.ds(ebase + t * cs, cs)], cbig)

            def chunk(k, carry1):
                off = k * chk
                for q in range(chk // _L):
                    sl = pl.ds(q * _L, _L)
                    ridx[sl] = rbig[pl.ds(off + q * _L, _L)]
                    cidx[sl] = cbig[pl.ds(off + q * _L, _L)] + noff
                pltpu.sync_copy(ht_hbm.at[cidx], gbuf)
                woff = t * cs + off

                def rowgrp(rg, carry2):
                    base = rg * 8
                    for u in range(8):
                        r = base + u
                        wv = plsc.load_gather(
                            w_v, [jnp.full((_L,), woff + r, jnp.int32)])
                        for j in range(hh // _L):
                            sl = pl.ds(j * _L, _L)
                            gbuf[r, sl] = gbuf[r, sl] * wv
                    return carry2

                lax.fori_loop(0, chk // 8, rowgrp, 0)
                pltpu.sync_copy(gbuf, agg.at[ridx], add=True)
                return carry1

            lax.fori_loop(0, cs // chk, chunk, 0)
            return carry

        lax.fori_loop(0, nst, stage_c, 0)

        plsc.subcore_barrier()
        pltpu.sync_copy(agg.at[pl.ds(s * nzr, nzr)],
                        out_hbm.at[pl.ds(c * n + s * nzr, nzr)])

    return edge_k


def kernel(x, edge_index, Wp, bp, W0, a0, W1, a1, Wh1, bh1, Wh2, bh2):
    n, d = x.shape
    e = edge_index.shape[1]
    h = Wp.shape[0]
    hh = h // 2
    co = Wh2.shape[0]
    bn = 1000 if n % 1000 == 0 else n
    grid = (n // bn,)
    f32 = jnp.float32

    edge_k = _make_edge_kernel(n, e, hh)
    zeros_h = jnp.zeros((n, hh), f32)
    mm_outs = [
        jax.ShapeDtypeStruct((2, n, hh), f32),
        jax.ShapeDtypeStruct((n, 2), f32),
    ]
    mm_out_specs = [
        pl.BlockSpec((2, bn, hh), lambda i: (0, i, 0)),
        pl.BlockSpec((bn, 2), lambda i: (i, 0)),
    ]

    a2_0 = jnp.stack([a0[:h], a0[h:]], axis=1)
    ht0, s20 = pl.pallas_call(
        _fused0_body,
        grid=grid,
        in_specs=[
            pl.BlockSpec((bn, d), lambda i: (i, 0)),
            pl.BlockSpec((d, h), lambda i: (0, 0)),
            pl.BlockSpec((1, h), lambda i: (0, 0)),
            pl.BlockSpec((h, h), lambda i: (0, 0)),
            pl.BlockSpec((h, 2), lambda i: (0, 0)),
        ],
        out_specs=mm_out_specs,
        out_shape=mm_outs,
    )(x, Wp.T, bp[None, :], W0, a2_0)
    agg0 = edge_k(ht0.reshape(2 * n, hh), s20.reshape(2 * n),
                  edge_index, zeros_h).reshape(2, n, hh)

    a2_1 = jnp.stack([a1[:h], a1[h:]], axis=1)
    ht1, s21 = pl.pallas_call(
        _layer_mm_body,
        grid=grid,
        in_specs=[
            pl.BlockSpec((1, bn, hh), lambda i: (0, i, 0)),
            pl.BlockSpec((1, bn, hh), lambda i: (1, i, 0)),
            pl.BlockSpec((h, h), lambda i: (0, 0)),
            pl.BlockSpec((h, 2), lambda i: (0, 0)),
        ],
        out_specs=mm_out_specs,
        out_shape=mm_outs,
    )(agg0, agg0, W1, a2_1)
    p2 = edge_k(ht1.reshape(2 * n, hh), s21.reshape(2 * n),
                edge_index, zeros_h).reshape(2, n, hh)

    out = pl.pallas_call(
        _head_body,
        grid=grid,
        in_specs=[
            pl.BlockSpec((1, bn, hh), lambda i: (0, i, 0)),
            pl.BlockSpec((1, bn, hh), lambda i: (1, i, 0)),
            pl.BlockSpec((h, hh), lambda i: (0, 0)),
            pl.BlockSpec((1, hh), lambda i: (0, 0)),
            pl.BlockSpec((hh, co), lambda i: (0, 0)),
            pl.BlockSpec((1, co), lambda i: (0, 0)),
        ],
        out_specs=pl.BlockSpec((bn, co), lambda i: (i, 0)),
        out_shape=jax.ShapeDtypeStruct((n, co), f32),
    )(p2, p2, Wh1.T, bh1[None, :], Wh2.T, bh2[None, :])
    return out


# speedup vs baseline: 4.4742x; 1.0097x over previous
import functools

import jax
import jax.numpy as jnp
from jax import lax
from jax.experimental import pallas as pl
from jax.experimental.pallas import tpu as pltpu
from jax.experimental.pallas import tpu_sc as plsc

_NS = 16
_L = 16


def _elu(v):
    return jnp.where(v > 0, v, jnp.exp(v) - 1.0)


def _fused0_body(x_ref, wpt_ref, bp_ref, w_ref, a2_ref, ht_ref, s2_ref):
    p = jnp.dot(x_ref[...], wpt_ref[...], preferred_element_type=jnp.float32)
    hact = _elu(p + bp_ref[...])
    ht = jnp.dot(hact, w_ref[...], preferred_element_type=jnp.float32)
    hh = ht.shape[1] // 2
    ht_ref[0] = ht[:, :hh]
    ht_ref[1] = ht[:, hh:]
    s2_ref[...] = jnp.dot(ht, a2_ref[...], preferred_element_type=jnp.float32)


def _layer_mm_body(pa_ref, pb_ref, w_ref, a2_ref, ht_ref, s2_ref):
    hact = _elu(jnp.concatenate([pa_ref[0], pb_ref[0]], axis=-1))
    ht = jnp.dot(hact, w_ref[...], preferred_element_type=jnp.float32)
    hh = ht.shape[1] // 2
    ht_ref[0] = ht[:, :hh]
    ht_ref[1] = ht[:, hh:]
    s2_ref[...] = jnp.dot(ht, a2_ref[...], preferred_element_type=jnp.float32)


def _head_body(pa_ref, pb_ref, w1t_ref, b1_ref, w2t_ref, b2_ref, o_ref):
    hact = _elu(jnp.concatenate([pa_ref[0], pb_ref[0]], axis=-1))
    z = jnp.dot(hact, w1t_ref[...], preferred_element_type=jnp.float32)
    z = jnp.maximum(z + b1_ref[...], 0.0)
    o = jnp.dot(z, w2t_ref[...], preferred_element_type=jnp.float32)
    o_ref[...] = o + b2_ref[...]


@functools.lru_cache(maxsize=None)
def _make_edge_kernel(n, e, hh):
    ept = e // _NS
    nzr = n // _NS
    chk = 80
    cs = 2000
    nst = ept // cs
    ng = ept // _L
    assert n % _NS == 0 and ept % cs == 0 and cs % chk == 0 and cs % _L == 0

    mesh = plsc.VectorSubcoreMesh(
        core_axis_name="c", subcore_axis_name="s",
        num_cores=2, num_subcores=_NS)

    @functools.partial(
        pl.kernel,
        out_type=jax.ShapeDtypeStruct((2 * n, hh), jnp.float32),
        mesh=mesh,
        compiler_params=pltpu.CompilerParams(
            use_tc_tiling_on_sc=False, needs_layout_passes=False),
        scratch_types=[
            pltpu.VMEM((2 * n,), jnp.float32),
            pltpu.VMEM((ept,), jnp.float32),
            pltpu.VMEM((cs,), jnp.int32),
            pltpu.VMEM((cs,), jnp.int32),
            pltpu.VMEM((chk, hh), jnp.float32),
            pltpu.VMEM((chk,), jnp.int32),
            pltpu.VMEM((chk,), jnp.int32),
            pltpu.VMEM((_L,), jnp.float32),
            pltpu.VMEM((_NS, _L), jnp.float32),
            pltpu.VMEM_SHARED((_NS, _L), jnp.float32),
            pltpu.VMEM_SHARED((_NS, _L), jnp.float32),
            pltpu.VMEM_SHARED((n, hh), jnp.float32),
        ],
    )
    def edge_k(ht_hbm, s2_hbm, eix_hbm, zeros_hbm, out_hbm,
               s2_v, w_v, rbig, cbig, gbuf, ridx, cidx, statb, red_v,
               stage_mx, stage_sm, agg):
        c = lax.axis_index("c")
        s = lax.axis_index("s")
        ebase = s * ept

        pltpu.sync_copy(zeros_hbm.at[pl.ds(s * nzr, nzr)],
                        agg.at[pl.ds(s * nzr, nzr)])

        pltpu.sync_copy(s2_hbm, s2_v)

        o16 = jnp.ones((_L,), jnp.int32)
        noff = c * n

        def stage_a(t, mx):
            pltpu.sync_copy(eix_hbm.at[0, pl.ds(ebase + t * cs, cs)], rbig)
            pltpu.sync_copy(eix_hbm.at[1, pl.ds(ebase + t * cs, cs)], cbig)

            def body_a(g, mx2):
                sl = pl.ds(g * _L, _L)
                sr = plsc.load_gather(s2_v, [rbig[sl] * 2])
                sc = plsc.load_gather(s2_v, [cbig[sl] * 2 + o16])
                al = sr + sc
                al = jnp.where(al > 0, al, al * 0.01)
                w_v[pl.ds(t * cs + g * _L, _L)] = al
                return jnp.maximum(mx2, al)

            return lax.fori_loop(0, cs // _L, body_a, mx)

        mx = lax.fori_loop(0, nst, stage_a,
                           jnp.full((_L,), -3.0e38, jnp.float32))
        statb[...] = mx
        pltpu.sync_copy(statb, stage_mx.at[s])
        plsc.subcore_barrier()
        pltpu.sync_copy(stage_mx, red_v)
        acc = red_v[0]
        for i in range(1, _NS):
            acc = jnp.maximum(acc, red_v[i])
        m = jnp.max(acc)

        def body_b(g, sm):
            sl = pl.ds(g * _L, _L)
            ex = jnp.exp(w_v[sl] - m)
            w_v[sl] = ex
            return sm + ex

        sm = lax.fori_loop(0, ng, body_b, jnp.zeros((_L,), jnp.float32))
        statb[...] = sm
        pltpu.sync_copy(statb, stage_sm.at[s])
        plsc.subcore_barrier()
        pltpu.sync_copy(stage_sm, red_v)
        acc = red_v[0]
        for i in range(1, _NS):
            acc = acc + red_v[i]
        invv = jnp.ones((_L,), jnp.float32) / jnp.broadcast_to(
            jnp.sum(acc), (_L,))

        def body_c(g, carry):
            sl = pl.ds(g * _L, _L)
            w_v[sl] = w_v[sl] * invv
            return carry

        lax.fori_loop(0, ng, body_c, 0)

        def stage_c(t, carry):
            pltpu.sync_copy(eix_hbm.at[0, pl.ds(ebase + t * cs, cs)], rbig)
            pltpu.sync_copy(eix_hbm.at[1, pl.ds(ebase + t * cs, cs)], cbig)

            def chunk(k, carry1):
                off = k * chk
                for q in range(chk // _L):
                    sl = pl.ds(q * _L, _L)
                    ridx[sl] = rbig[pl.ds(off + q * _L, _L)]
                    cidx[sl] = cbig[pl.ds(off + q * _L, _L)] + noff
                pltpu.sync_copy(ht_hbm.at[cidx], gbuf)
                woff = t * cs + off

                def rowgrp(rg, carry2):
                    base = rg * 8
                    for u in range(8):
                        r = base + u
                        wv = plsc.load_gather(
                            w_v, [jnp.full((_L,), woff + r, jnp.int32)])
                        for j in range(hh // _L):
                            sl = pl.ds(j * _L, _L)
                            gbuf[r, sl] = gbuf[r, sl] * wv
                    return carry2

                lax.fori_loop(0, chk // 8, rowgrp, 0)
                pltpu.sync_copy(gbuf, agg.at[ridx], add=True)
                return carry1

            lax.fori_loop(0, cs // chk, chunk, 0)
            return carry

        lax.fori_loop(0, nst, stage_c, 0)

        plsc.subcore_barrier()
        pltpu.sync_copy(agg.at[pl.ds(s * nzr, nzr)],
                        out_hbm.at[pl.ds(c * n + s * nzr, nzr)])

    return edge_k


def kernel(x, edge_index, Wp, bp, W0, a0, W1, a1, Wh1, bh1, Wh2, bh2):
    n, d = x.shape
    e = edge_index.shape[1]
    h = Wp.shape[0]
    hh = h // 2
    co = Wh2.shape[0]
    bn = 2000 if n % 2000 == 0 else n
    grid = (n // bn,)
    f32 = jnp.float32

    edge_k = _make_edge_kernel(n, e, hh)
    zeros_h = jnp.zeros((n, hh), f32)
    mm_outs = [
        jax.ShapeDtypeStruct((2, n, hh), f32),
        jax.ShapeDtypeStruct((n, 2), f32),
    ]
    mm_out_specs = [
        pl.BlockSpec((2, bn, hh), lambda i: (0, i, 0)),
        pl.BlockSpec((bn, 2), lambda i: (i, 0)),
    ]

    a2_0 = jnp.stack([a0[:h], a0[h:]], axis=1)
    ht0, s20 = pl.pallas_call(
        _fused0_body,
        grid=grid,
        in_specs=[
            pl.BlockSpec((bn, d), lambda i: (i, 0)),
            pl.BlockSpec((d, h), lambda i: (0, 0)),
            pl.BlockSpec((1, h), lambda i: (0, 0)),
            pl.BlockSpec((h, h), lambda i: (0, 0)),
            pl.BlockSpec((h, 2), lambda i: (0, 0)),
        ],
        out_specs=mm_out_specs,
        out_shape=mm_outs,
    )(x, Wp.T, bp[None, :], W0, a2_0)
    agg0 = edge_k(ht0.reshape(2 * n, hh), s20.reshape(2 * n),
                  edge_index, zeros_h).reshape(2, n, hh)

    a2_1 = jnp.stack([a1[:h], a1[h:]], axis=1)
    ht1, s21 = pl.pallas_call(
        _layer_mm_body,
        grid=grid,
        in_specs=[
            pl.BlockSpec((1, bn, hh), lambda i: (0, i, 0)),
            pl.BlockSpec((1, bn, hh), lambda i: (1, i, 0)),
            pl.BlockSpec((h, h), lambda i: (0, 0)),
            pl.BlockSpec((h, 2), lambda i: (0, 0)),
        ],
        out_specs=mm_out_specs,
        out_shape=mm_outs,
    )(agg0, agg0, W1, a2_1)
    p2 = edge_k(ht1.reshape(2 * n, hh), s21.reshape(2 * n),
                edge_index, zeros_h).reshape(2, n, hh)

    out = pl.pallas_call(
        _head_body,
        grid=grid,
        in_specs=[
            pl.BlockSpec((1, bn, hh), lambda i: (0, i, 0)),
            pl.BlockSpec((1, bn, hh), lambda i: (1, i, 0)),
            pl.BlockSpec((h, hh), lambda i: (0, 0)),
            pl.BlockSpec((1, hh), lambda i: (0, 0)),
            pl.BlockSpec((hh, co), lambda i: (0, 0)),
            pl.BlockSpec((1, co), lambda i: (0, 0)),
        ],
        out_specs=pl.BlockSpec((bn, co), lambda i: (i, 0)),
        out_shape=jax.ShapeDtypeStruct((n, co), f32),
    )(p2, p2, Wh1.T, bh1[None, :], Wh2.T, bh2[None, :])
    return out


# speedup vs baseline: 5.1046x; 1.1409x over previous
import functools

import jax
import jax.numpy as jnp
from jax import lax
from jax.experimental import pallas as pl
from jax.experimental.pallas import tpu as pltpu
from jax.experimental.pallas import tpu_sc as plsc

_NS = 16
_L = 16


def _elu(v):
    return jnp.where(v > 0, v, jnp.exp(v) - 1.0)


def _fused0_body(x_ref, wpt_ref, bp_ref, w_ref, a2_ref, ht_ref, s2_ref):
    p = jnp.dot(x_ref[...], wpt_ref[...], preferred_element_type=jnp.float32)
    hact = _elu(p + bp_ref[...])
    ht = jnp.dot(hact, w_ref[...], preferred_element_type=jnp.float32)
    hh = ht.shape[1] // 2
    ht_ref[0] = ht[:, :hh]
    ht_ref[1] = ht[:, hh:]
    s2_ref[...] = jnp.dot(ht, a2_ref[...], preferred_element_type=jnp.float32)


def _layer_mm_body(pa_ref, pb_ref, w_ref, a2_ref, ht_ref, s2_ref):
    hact = _elu(jnp.concatenate([pa_ref[0], pb_ref[0]], axis=-1))
    ht = jnp.dot(hact, w_ref[...], preferred_element_type=jnp.float32)
    hh = ht.shape[1] // 2
    ht_ref[0] = ht[:, :hh]
    ht_ref[1] = ht[:, hh:]
    s2_ref[...] = jnp.dot(ht, a2_ref[...], preferred_element_type=jnp.float32)


def _head_body(pa_ref, pb_ref, w1t_ref, b1_ref, w2t_ref, b2_ref, o_ref):
    hact = _elu(jnp.concatenate([pa_ref[0], pb_ref[0]], axis=-1))
    z = jnp.dot(hact, w1t_ref[...], preferred_element_type=jnp.float32)
    z = jnp.maximum(z + b1_ref[...], 0.0)
    o = jnp.dot(z, w2t_ref[...], preferred_element_type=jnp.float32)
    o_ref[...] = o + b2_ref[...]


@functools.lru_cache(maxsize=None)
def _make_edge_kernel(n, e, hh):
    ept = e // _NS
    nzr = n // _NS
    chk = 128
    ng = ept // _L
    assert n % _NS == 0 and ept % chk == _L and ept % _L == 0

    mesh = plsc.VectorSubcoreMesh(
        core_axis_name="c", subcore_axis_name="s",
        num_cores=2, num_subcores=_NS)

    @functools.partial(
        pl.kernel,
        out_type=jax.ShapeDtypeStruct((2 * n, hh), jnp.float32),
        mesh=mesh,
        compiler_params=pltpu.CompilerParams(
            use_tc_tiling_on_sc=False, needs_layout_passes=False),
        scratch_types=[
            pltpu.VMEM((ept,), jnp.float32),
            pltpu.VMEM((ept,), jnp.int32),
            pltpu.VMEM((ept,), jnp.int32),
            pltpu.VMEM((_L,), jnp.float32),
            pltpu.VMEM((_NS, _L), jnp.float32),
            pltpu.VMEM_SHARED((_NS, _L), jnp.float32),
            pltpu.VMEM_SHARED((_NS, _L), jnp.float32),
            pltpu.VMEM_SHARED((n, hh), jnp.float32),
        ],
    )
    def edge_k(ht_hbm, s2_hbm, eix_hbm, zeros_hbm, out_hbm,
               w_v, rbig, cbig, statb, red_v,
               stage_mx, stage_sm, agg):
        c = lax.axis_index("c")
        s = lax.axis_index("s")
        ebase = s * ept

        pltpu.sync_copy(zeros_hbm.at[pl.ds(s * nzr, nzr)],
                        agg.at[pl.ds(s * nzr, nzr)])

        pltpu.sync_copy(eix_hbm.at[0, pl.ds(ebase, ept)], rbig)
        pltpu.sync_copy(eix_hbm.at[1, pl.ds(ebase, ept)], cbig)

        o16 = jnp.ones((_L,), jnp.int32)
        noff = c * n

        def phases_ab(s2_v):
            pltpu.sync_copy(s2_hbm, s2_v)

            def body_a(g, mx2):
                sl = pl.ds(g * _L, _L)
                sr = plsc.load_gather(s2_v, [rbig[sl] * 2])
                sc = plsc.load_gather(s2_v, [cbig[sl] * 2 + o16])
                al = sr + sc
                al = jnp.where(al > 0, al, al * 0.01)
                w_v[sl] = al
                return jnp.maximum(mx2, al)

            mx = lax.fori_loop(0, ng, body_a,
                               jnp.full((_L,), -3.0e38, jnp.float32))
            statb[...] = mx
            pltpu.sync_copy(statb, stage_mx.at[s])
            plsc.subcore_barrier()
            pltpu.sync_copy(stage_mx, red_v)
            acc = red_v[0]
            for i in range(1, _NS):
                acc = jnp.maximum(acc, red_v[i])
            m = jnp.max(acc)

            def body_b(g, sm):
                sl = pl.ds(g * _L, _L)
                ex = jnp.exp(w_v[sl] - m)
                w_v[sl] = ex
                return sm + ex

            sm = lax.fori_loop(0, ng, body_b, jnp.zeros((_L,), jnp.float32))
            statb[...] = sm
            pltpu.sync_copy(statb, stage_sm.at[s])
            plsc.subcore_barrier()
            pltpu.sync_copy(stage_sm, red_v)
            acc = red_v[0]
            for i in range(1, _NS):
                acc = acc + red_v[i]
            invv = jnp.ones((_L,), jnp.float32) / jnp.broadcast_to(
                jnp.sum(acc), (_L,))

            def body_n(g, carry):
                sl = pl.ds(g * _L, _L)
                w_v[sl] = w_v[sl] * invv
                return carry

            lax.fori_loop(0, ng, body_n, 0)

        pl.run_scoped(phases_ab, pltpu.VMEM((2 * n,), jnp.float32))

        nch = ept // chk

        def scale_rows(gref, woff, nrows):
            def rowgrp(rg, carry2):
                base = rg * 8
                for u in range(8):
                    r = base + u
                    wv = plsc.load_gather(
                        w_v, [jnp.full((_L,), woff + r, jnp.int32)])
                    for j in range(hh // _L):
                        sl = pl.ds(j * _L, _L)
                        gref[r, sl] = gref[r, sl] * wv
                return carry2

            lax.fori_loop(0, nrows // 8, rowgrp, 0)

        def phase_c(gbuf, ridx, cidx, gtl, rtl, ctl):
            def chunk(k, carry1):
                off = k * chk
                for q in range(chk // _L):
                    sl = pl.ds(q * _L, _L)
                    src = pl.ds(off + q * _L, _L)
                    ridx[sl] = rbig[src]
                    cidx[sl] = cbig[src] + noff
                pltpu.sync_copy(ht_hbm.at[cidx], gbuf)
                scale_rows(gbuf, off, chk)
                pltpu.sync_copy(gbuf, agg.at[ridx], add=True)
                return carry1

            lax.fori_loop(0, nch, chunk, 0)

            toff = nch * chk
            rtl[...] = rbig[pl.ds(toff, _L)]
            ctl[...] = cbig[pl.ds(toff, _L)] + noff
            pltpu.sync_copy(ht_hbm.at[ctl], gtl)
            scale_rows(gtl, toff, _L)
            pltpu.sync_copy(gtl, agg.at[rtl], add=True)

        pl.run_scoped(phase_c,
                      pltpu.VMEM((chk, hh), jnp.float32),
                      pltpu.VMEM((chk,), jnp.int32),
                      pltpu.VMEM((chk,), jnp.int32),
                      pltpu.VMEM((_L, hh), jnp.float32),
                      pltpu.VMEM((_L,), jnp.int32),
                      pltpu.VMEM((_L,), jnp.int32))

        plsc.subcore_barrier()
        pltpu.sync_copy(agg.at[pl.ds(s * nzr, nzr)],
                        out_hbm.at[pl.ds(c * n + s * nzr, nzr)])

    return edge_k


def kernel(x, edge_index, Wp, bp, W0, a0, W1, a1, Wh1, bh1, Wh2, bh2):
    n, d = x.shape
    e = edge_index.shape[1]
    h = Wp.shape[0]
    hh = h // 2
    co = Wh2.shape[0]
    bn = 2000 if n % 2000 == 0 else n
    grid = (n // bn,)
    f32 = jnp.float32

    edge_k = _make_edge_kernel(n, e, hh)
    zeros_h = jnp.zeros((n, hh), f32)
    mm_outs = [
        jax.ShapeDtypeStruct((2, n, hh), f32),
        jax.ShapeDtypeStruct((n, 2), f32),
    ]
    mm_out_specs = [
        pl.BlockSpec((2, bn, hh), lambda i: (0, i, 0)),
        pl.BlockSpec((bn, 2), lambda i: (i, 0)),
    ]

    a2_0 = jnp.stack([a0[:h], a0[h:]], axis=1)
    ht0, s20 = pl.pallas_call(
        _fused0_body,
        grid=grid,
        in_specs=[
            pl.BlockSpec((bn, d), lambda i: (i, 0)),
            pl.BlockSpec((d, h), lambda i: (0, 0)),
            pl.BlockSpec((1, h), lambda i: (0, 0)),
            pl.BlockSpec((h, h), lambda i: (0, 0)),
            pl.BlockSpec((h, 2), lambda i: (0, 0)),
        ],
        out_specs=mm_out_specs,
        out_shape=mm_outs,
    )(x, Wp.T, bp[None, :], W0, a2_0)
    agg0 = edge_k(ht0.reshape(2 * n, hh), s20.reshape(2 * n),
                  edge_index, zeros_h).reshape(2, n, hh)

    a2_1 = jnp.stack([a1[:h], a1[h:]], axis=1)
    ht1, s21 = pl.pallas_call(
        _layer_mm_body,
        grid=grid,
        in_specs=[
            pl.BlockSpec((1, bn, hh), lambda i: (0, i, 0)),
            pl.BlockSpec((1, bn, hh), lambda i: (1, i, 0)),
            pl.BlockSpec((h, h), lambda i: (0, 0)),
            pl.BlockSpec((h, 2), lambda i: (0, 0)),
        ],
        out_specs=mm_out_specs,
        out_shape=mm_outs,
    )(agg0, agg0, W1, a2_1)
    p2 = edge_k(ht1.reshape(2 * n, hh), s21.reshape(2 * n),
                edge_index, zeros_h).reshape(2, n, hh)

    out = pl.pallas_call(
        _head_body,
        grid=grid,
        in_specs=[
            pl.BlockSpec((1, bn, hh), lambda i: (0, i, 0)),
            pl.BlockSpec((1, bn, hh), lambda i: (1, i, 0)),
            pl.BlockSpec((h, hh), lambda i: (0, 0)),
            pl.BlockSpec((1, hh), lambda i: (0, 0)),
            pl.BlockSpec((hh, co), lambda i: (0, 0)),
            pl.BlockSpec((1, co), lambda i: (0, 0)),
        ],
        out_specs=pl.BlockSpec((bn, co), lambda i: (i, 0)),
        out_shape=jax.ShapeDtypeStruct((n, co), f32),
    )(p2, p2, Wh1.T, bh1[None, :], Wh2.T, bh2[None, :])
    return out
